# baseline (speedup 1.0000x reference)
import functools

import jax
import jax.numpy as jnp
from jax import lax
from jax.experimental import pallas as pl
from jax.experimental.pallas import tpu as pltpu
from jax.experimental.pallas import tpu_sc as plsc

T = 2048
D = 768
E = 8
H = 4 * D
TILE_M = 256
TILES = T // TILE_M + E - 1
RP = TILES * TILE_M
NBLK = 512
NB = H // NBLK
NW = 32
BPW = RP // NW
OUT_PAD = T + 8


def _router_body(x_ref, wr_ref, o_ref):
    logits = jnp.dot(x_ref[...], wr_ref[...], preferred_element_type=jnp.float32)
    m = jnp.max(logits, axis=1, keepdims=True)
    col = lax.broadcasted_iota(jnp.int32, logits.shape, 1)
    idx = jnp.min(jnp.where(logits >= m, col, E), axis=1, keepdims=True)
    o_ref[...] = jnp.broadcast_to(idx, o_ref.shape)


def _router(x, Wr):
    out = pl.pallas_call(
        _router_body,
        out_shape=jax.ShapeDtypeStruct((T, E), jnp.int32),
    )(x, Wr)
    return out[:, 0]


def _mlp_body(eof_ref, nmap_ref, valid_ref, xs_ref, wg_ref, wu_ref, cp_ref,
              o_ref, acc_ref):
    s = pl.program_id(0)
    n = pl.program_id(1)

    @pl.when(valid_ref[s] == 1)
    def _():
        xt = xs_ref[...]
        wg = wg_ref[0].astype(jnp.bfloat16)
        wu = wu_ref[0].astype(jnp.bfloat16)
        cp = cp_ref[0].astype(jnp.bfloat16)
        g = jnp.dot(xt, wg, preferred_element_type=jnp.float32)
        u = jnp.dot(xt, wu, preferred_element_type=jnp.float32)
        a = (g * jax.nn.sigmoid(g) * u).astype(jnp.bfloat16)
        part = jnp.dot(a, cp, preferred_element_type=jnp.float32)

        @pl.when(n == 0)
        def _():
            acc_ref[...] = part

        @pl.when(n > 0)
        def _():
            acc_ref[...] += part

        @pl.when(n == NB - 1)
        def _():
            o_ref[...] = acc_ref[...].astype(jnp.bfloat16)


def _grouped_mlp(xs, w_v, c_proj, eof, nmap, valid):
    grid_spec = pltpu.PrefetchScalarGridSpec(
        num_scalar_prefetch=3,
        grid=(TILES, NB),
        in_specs=[
            pl.BlockSpec((TILE_M, D), lambda s, n, eof, nmap, valid: (s, 0)),
            pl.BlockSpec((1, D, NBLK),
                         lambda s, n, eof, nmap, valid:
                         (eof[s], 0, nmap[s, n])),
            pl.BlockSpec((1, D, NBLK),
                         lambda s, n, eof, nmap, valid:
                         (eof[s], 0, NB + nmap[s, n])),
            pl.BlockSpec((1, NBLK, D),
                         lambda s, n, eof, nmap, valid:
                         (eof[s], nmap[s, n], 0)),
        ],
        out_specs=pl.BlockSpec((TILE_M, D),
                               lambda s, n, eof, nmap, valid: (s, 0)),
        scratch_shapes=[pltpu.VMEM((TILE_M, D), jnp.float32)],
    )
    return pl.pallas_call(
        _mlp_body,
        grid_spec=grid_spec,
        out_shape=jax.ShapeDtypeStruct((RP, D), jnp.bfloat16),
        compiler_params=pltpu.CompilerParams(
            dimension_semantics=("arbitrary", "arbitrary")),
    )(eof, nmap, valid, xs, w_v, w_v, c_proj)


D2 = D // 2


def _pack32(a):
    n, d = a.shape
    return lax.bitcast_convert_type(a.reshape(n, d // 2, 2), jnp.int32)


def _unpack32(a):
    n, d2 = a.shape
    return lax.bitcast_convert_type(a, jnp.bfloat16).reshape(n, 2 * d2)


def _sc_gather(xb, gidx):
    mesh = plsc.VectorSubcoreMesh(core_axis_name="c", subcore_axis_name="s")

    @functools.partial(
        pl.kernel,
        mesh=mesh,
        out_type=jax.ShapeDtypeStruct((RP, D2), jnp.int32),
        scratch_types=[
            pltpu.VMEM((BPW,), jnp.int32),
            pltpu.VMEM((BPW, D2), jnp.int32),
            pltpu.SemaphoreType.DMA,
        ],
    )
    def k(x_hbm, idx_hbm, out_hbm, idx_v, rows_v, sem):
        wid = lax.axis_index("s") * 2 + lax.axis_index("c")
        base = wid * BPW
        pltpu.sync_copy(idx_hbm.at[pl.ds(base, BPW)], idx_v)
        pltpu.async_copy(x_hbm.at[idx_v], rows_v, sem).wait()
        pltpu.sync_copy(rows_v, out_hbm.at[pl.ds(base, BPW)])

    return k(xb, gidx)


def _sc_scatter(y, sidx):
    mesh = plsc.VectorSubcoreMesh(core_axis_name="c", subcore_axis_name="s")

    @functools.partial(
        pl.kernel,
        mesh=mesh,
        out_type=jax.ShapeDtypeStruct((OUT_PAD, D2), jnp.int32),
        scratch_types=[
            pltpu.VMEM((BPW,), jnp.int32),
            pltpu.VMEM((BPW, D2), jnp.int32),
            pltpu.SemaphoreType.DMA,
        ],
    )
    def k(y_hbm, idx_hbm, out_hbm, idx_v, rows_v, sem):
        wid = lax.axis_index("s") * 2 + lax.axis_index("c")
        base = wid * BPW
        pltpu.sync_copy(idx_hbm.at[pl.ds(base, BPW)], idx_v)
        pltpu.sync_copy(y_hbm.at[pl.ds(base, BPW)], rows_v)
        pltpu.async_copy(rows_v, out_hbm.at[idx_v], sem).wait()

    return k(y, sidx)


def _schedule(eidx):
    order = jnp.argsort(eidx).astype(jnp.int32)
    cnt = jnp.sum(eidx[None, :] == jnp.arange(E, dtype=eidx.dtype)[:, None],
                  axis=1).astype(jnp.int32)
    offs = (jnp.cumsum(cnt) - cnt).astype(jnp.int32)
    tpe = (cnt + TILE_M - 1) // TILE_M
    incl = jnp.cumsum(tpe).astype(jnp.int32)
    tbase = incl - tpe
    total = incl[-1]

    t = jnp.arange(TILES, dtype=jnp.int32)
    e_raw = jnp.searchsorted(incl, t, side="right").astype(jnp.int32)
    e_last = jnp.searchsorted(incl, total - 1, side="right").astype(jnp.int32)
    validt = t < total
    eof = jnp.where(validt, jnp.clip(e_raw, 0, E - 1), e_last)
    tin = t - tbase[eof]
    serp = (tin % 2).astype(jnp.int32)

    narr = jnp.broadcast_to(jnp.arange(NB, dtype=jnp.int32)[None, :],
                            (TILES, NB))
    last_serp = (total - 1 - tbase[e_last]) % 2
    pin = jnp.where(last_serp == 0, NB - 1, 0).astype(jnp.int32)
    nmap = jnp.where(validt[:, None],
                     jnp.where(serp[:, None] == 1, NB - 1 - narr, narr),
                     pin)
    valid = validt.astype(jnp.int32)

    r = jnp.arange(RP, dtype=jnp.int32)
    er = eof[r // TILE_M]
    j = r - tbase[er] * TILE_M
    validr = j < cnt[er]
    src = order[jnp.clip(offs[er] + j, 0, T - 1)]
    gidx = jnp.where(validr, src, 0).astype(jnp.int32)
    sidx = jnp.where(validr, src, T).astype(jnp.int32)
    return eof, nmap, valid, gidx, sidx


def kernel(x, Wr, w_v, c_proj):
    eidx = _router(x, Wr)
    eof, nmap, valid, gidx, sidx = _schedule(eidx)
    xs = _sc_gather(_pack32(x.astype(jnp.bfloat16)), gidx)
    y = _grouped_mlp(_unpack32(xs), w_v, c_proj, eof, nmap, valid)
    outp = _sc_scatter(_pack32(y), sidx)
    return _unpack32(outp[:T]).astype(jnp.float32)


# speedup vs baseline: 2.9018x; 2.9018x over previous
import functools

import jax
import jax.numpy as jnp
from jax import lax
from jax.experimental import pallas as pl
from jax.experimental.pallas import tpu as pltpu
from jax.experimental.pallas import tpu_sc as plsc

T = 2048
D = 768
E = 8
H = 4 * D
TILE_M = 256
NT = T // TILE_M
STEPS = NT + E - 1
NBLK = 512
NB = H // NBLK
NW = 32
BPW = T // NW


def _router_body(x_ref, wr_ref, o_ref):
    logits = jnp.dot(x_ref[...], wr_ref[...], preferred_element_type=jnp.float32)
    m = jnp.max(logits, axis=1, keepdims=True)
    col = lax.broadcasted_iota(jnp.int32, logits.shape, 1)
    idx = jnp.min(jnp.where(logits >= m, col, E), axis=1, keepdims=True)
    o_ref[...] = jnp.broadcast_to(idx, o_ref.shape)


def _router(x, Wr):
    out = pl.pallas_call(
        _router_body,
        out_shape=jax.ShapeDtypeStruct((T, E), jnp.int32),
    )(x, Wr)
    return out[:, 0]


def _mlp_body(tile_ref, eof_ref, lo_ref, hi_ref, valid_ref, first_ref,
              nmap_ref, xs_ref, wg_ref, wu_ref, cp_ref, o_ref):
    s = pl.program_id(0)
    n = pl.program_id(1)

    @pl.when(valid_ref[s] == 1)
    def _():
        gid = tile_ref[s] * TILE_M + lax.broadcasted_iota(
            jnp.int32, (TILE_M, 1), 0)
        msk = ((gid >= lo_ref[s]) & (gid < hi_ref[s])).astype(jnp.bfloat16)
        xt = xs_ref[...].astype(jnp.bfloat16) * msk
        wg = wg_ref[0].astype(jnp.bfloat16)
        wu = wu_ref[0].astype(jnp.bfloat16)
        cp = cp_ref[0].astype(jnp.bfloat16)
        g = jnp.dot(xt, wg, preferred_element_type=jnp.float32)
        u = jnp.dot(xt, wu, preferred_element_type=jnp.float32)
        a = (g * jax.nn.sigmoid(g) * u).astype(jnp.bfloat16)
        part = jnp.dot(a, cp, preferred_element_type=jnp.float32)
        init = (n == 0) & (first_ref[s] == 1)

        @pl.when(init)
        def _():
            o_ref[...] = part

        @pl.when(jnp.logical_not(init))
        def _():
            o_ref[...] += part


def _grouped_mlp(xs, w_v, c_proj, sched):
    grid_spec = pltpu.PrefetchScalarGridSpec(
        num_scalar_prefetch=7,
        grid=(STEPS, NB),
        in_specs=[
            pl.BlockSpec((TILE_M, D),
                         lambda s, n, tile, eof, lo, hi, valid, first, nmap:
                         (tile[s], 0)),
            pl.BlockSpec((1, D, NBLK),
                         lambda s, n, tile, eof, lo, hi, valid, first, nmap:
                         (eof[s], 0, nmap[s, n])),
            pl.BlockSpec((1, D, NBLK),
                         lambda s, n, tile, eof, lo, hi, valid, first, nmap:
                         (eof[s], 0, NB + nmap[s, n])),
            pl.BlockSpec((1, NBLK, D),
                         lambda s, n, tile, eof, lo, hi, valid, first, nmap:
                         (eof[s], nmap[s, n], 0)),
        ],
        out_specs=pl.BlockSpec((TILE_M, D),
                               lambda s, n, tile, eof, lo, hi, valid, first,
                               nmap: (tile[s], 0)),
    )
    return pl.pallas_call(
        _mlp_body,
        grid_spec=grid_spec,
        out_shape=jax.ShapeDtypeStruct((T, D), jnp.float32),
        compiler_params=pltpu.CompilerParams(
            dimension_semantics=("arbitrary", "arbitrary")),
    )(*sched, xs, w_v, w_v, c_proj)


def _sc_gather(x, gidx):
    mesh = plsc.VectorSubcoreMesh(core_axis_name="c", subcore_axis_name="s")

    @functools.partial(
        pl.kernel,
        mesh=mesh,
        out_type=jax.ShapeDtypeStruct((T, D), jnp.float32),
        scratch_types=[
            pltpu.VMEM((BPW,), jnp.int32),
            pltpu.VMEM((BPW, D), jnp.float32),
            pltpu.SemaphoreType.DMA,
        ],
    )
    def k(x_hbm, idx_hbm, out_hbm, idx_v, rows_v, sem):
        wid = lax.axis_index("s") * 2 + lax.axis_index("c")
        base = wid * BPW
        pltpu.sync_copy(idx_hbm.at[pl.ds(base, BPW)], idx_v)
        pltpu.async_copy(x_hbm.at[idx_v], rows_v, sem).wait()
        pltpu.sync_copy(rows_v, out_hbm.at[pl.ds(base, BPW)])

    return k(x, gidx)


def _sc_scatter(y, sidx):
    mesh = plsc.VectorSubcoreMesh(core_axis_name="c", subcore_axis_name="s")

    @functools.partial(
        pl.kernel,
        mesh=mesh,
        out_type=jax.ShapeDtypeStruct((T, D), jnp.float32),
        scratch_types=[
            pltpu.VMEM((BPW,), jnp.int32),
            pltpu.VMEM((BPW, D), jnp.float32),
            pltpu.SemaphoreType.DMA,
        ],
    )
    def k(y_hbm, idx_hbm, out_hbm, idx_v, rows_v, sem):
        wid = lax.axis_index("s") * 2 + lax.axis_index("c")
        base = wid * BPW
        pltpu.sync_copy(idx_hbm.at[pl.ds(base, BPW)], idx_v)
        pltpu.sync_copy(y_hbm.at[pl.ds(base, BPW)], rows_v)
        pltpu.async_copy(rows_v, out_hbm.at[idx_v], sem).wait()

    return k(y, sidx)


def _schedule(eidx):
    order = jnp.argsort(eidx).astype(jnp.int32)
    cnt = jnp.sum(eidx[None, :] == jnp.arange(E, dtype=eidx.dtype)[:, None],
                  axis=1).astype(jnp.int32)
    offs = (jnp.cumsum(cnt) - cnt).astype(jnp.int32)
    cum_end = offs + cnt

    cuts = jnp.sort(jnp.concatenate(
        [jnp.arange(NT, dtype=jnp.int32) * TILE_M, offs[1:]]))
    lo = cuts
    hi = jnp.concatenate([cuts[1:], jnp.full((1,), T, jnp.int32)])
    validv = hi > lo
    eraw = jnp.clip(jnp.searchsorted(cum_end, lo, side="right"),
                    0, E - 1).astype(jnp.int32)
    traw = jnp.clip(lo // TILE_M, 0, NT - 1)

    v = jnp.arange(STEPS, dtype=jnp.int32)
    prev = jnp.clip(lax.cummax(jnp.where(validv, v, -1)), 0, STEPS - 1)
    eof = jnp.where(validv, eraw, eraw[prev])
    tile = jnp.where(validv, traw, traw[prev])

    same_before = ((eraw[None, :] == eraw[:, None]) & validv[None, :]
                   & (v[None, :] < v[:, None]))
    dirn = (jnp.sum(same_before, axis=1) % 2).astype(jnp.int32)
    final_neff = jnp.where(dirn == 0, NB - 1, 0)
    narr = jnp.broadcast_to(jnp.arange(NB, dtype=jnp.int32)[None, :],
                            (STEPS, NB))
    nmap = jnp.where(validv[:, None],
                     jnp.where(dirn[:, None] == 1, NB - 1 - narr, narr),
                     final_neff[prev][:, None])
    first = (validv & (lo % TILE_M == 0)).astype(jnp.int32)
    valid = validv.astype(jnp.int32)
    return (tile, eof, lo, hi, valid, first, nmap), order


def kernel(x, Wr, w_v, c_proj):
    eidx = _router(x, Wr)
    sched, order = _schedule(eidx)
    xs = _sc_gather(x, order)
    y = _grouped_mlp(xs, w_v, c_proj, sched)
    return _sc_scatter(y, order)


# speedup vs baseline: 3.2550x; 1.1217x over previous
import functools

import jax
import jax.numpy as jnp
from jax import lax
from jax.experimental import pallas as pl
from jax.experimental.pallas import tpu as pltpu
from jax.experimental.pallas import tpu_sc as plsc

T = 2048
D = 768
E = 8
H = 4 * D
TILE_M = 256
NT = T // TILE_M
STEPS = NT + E - 1
NBLK = 768
NB = H // NBLK
NW = 32
BPW = T // NW


def _router_body(x_ref, wr_ref, o_ref):
    logits = jnp.dot(x_ref[...], wr_ref[...], preferred_element_type=jnp.float32)
    m = jnp.max(logits, axis=1, keepdims=True)
    col = lax.broadcasted_iota(jnp.int32, logits.shape, 1)
    idx = jnp.min(jnp.where(logits >= m, col, E), axis=1, keepdims=True)
    o_ref[...] = jnp.broadcast_to(idx, o_ref.shape)


def _router(x, Wr):
    out = pl.pallas_call(
        _router_body,
        out_shape=jax.ShapeDtypeStruct((T, E), jnp.int32),
    )(x, Wr)
    return out[:, 0]


def _mlp_body(tile_ref, eof_ref, lo_ref, hi_ref, valid_ref, first_ref,
              nmap_ref, xs_ref, wg_ref, wu_ref, cp_ref, o_ref):
    s = pl.program_id(0)
    n = pl.program_id(1)

    @pl.when(valid_ref[s] == 1)
    def _():
        gid = tile_ref[s] * TILE_M + lax.broadcasted_iota(
            jnp.int32, (TILE_M, 1), 0)
        msk = ((gid >= lo_ref[s]) & (gid < hi_ref[s])).astype(jnp.bfloat16)
        xt = xs_ref[...].astype(jnp.bfloat16) * msk
        wg = wg_ref[0].astype(jnp.bfloat16)
        wu = wu_ref[0].astype(jnp.bfloat16)
        cp = cp_ref[0].astype(jnp.bfloat16)
        g = jnp.dot(xt, wg, preferred_element_type=jnp.float32)
        u = jnp.dot(xt, wu, preferred_element_type=jnp.float32)
        a = (g * jax.nn.sigmoid(g) * u).astype(jnp.bfloat16)
        part = jnp.dot(a, cp, preferred_element_type=jnp.float32)
        init = (n == 0) & (first_ref[s] == 1)

        @pl.when(init)
        def _():
            o_ref[...] = part

        @pl.when(jnp.logical_not(init))
        def _():
            o_ref[...] += part


def _grouped_mlp(xs, w_v, c_proj, sched):
    grid_spec = pltpu.PrefetchScalarGridSpec(
        num_scalar_prefetch=7,
        grid=(STEPS, NB),
        in_specs=[
            pl.BlockSpec((TILE_M, D),
                         lambda s, n, tile, eof, lo, hi, valid, first, nmap:
                         (tile[s], 0)),
            pl.BlockSpec((1, D, NBLK),
                         lambda s, n, tile, eof, lo, hi, valid, first, nmap:
                         (eof[s], 0, nmap[s, n])),
            pl.BlockSpec((1, D, NBLK),
                         lambda s, n, tile, eof, lo, hi, valid, first, nmap:
                         (eof[s], 0, NB + nmap[s, n])),
            pl.BlockSpec((1, NBLK, D),
                         lambda s, n, tile, eof, lo, hi, valid, first, nmap:
                         (eof[s], nmap[s, n], 0)),
        ],
        out_specs=pl.BlockSpec((TILE_M, D),
                               lambda s, n, tile, eof, lo, hi, valid, first,
                               nmap: (tile[s], 0)),
    )
    return pl.pallas_call(
        _mlp_body,
        grid_spec=grid_spec,
        out_shape=jax.ShapeDtypeStruct((T, D), jnp.float32),
        compiler_params=pltpu.CompilerParams(
            dimension_semantics=("arbitrary", "arbitrary")),
    )(*sched, xs, w_v, w_v, c_proj)


def _sc_gather(x, gidx):
    mesh = plsc.VectorSubcoreMesh(core_axis_name="c", subcore_axis_name="s")

    @functools.partial(
        pl.kernel,
        mesh=mesh,
        out_type=jax.ShapeDtypeStruct((T, D), jnp.float32),
        scratch_types=[
            pltpu.VMEM((BPW,), jnp.int32),
            pltpu.VMEM((BPW, D), jnp.float32),
            pltpu.SemaphoreType.DMA,
        ],
    )
    def k(x_hbm, idx_hbm, out_hbm, idx_v, rows_v, sem):
        wid = lax.axis_index("s") * 2 + lax.axis_index("c")
        base = wid * BPW
        pltpu.sync_copy(idx_hbm.at[pl.ds(base, BPW)], idx_v)
        pltpu.async_copy(x_hbm.at[idx_v], rows_v, sem).wait()
        pltpu.sync_copy(rows_v, out_hbm.at[pl.ds(base, BPW)])

    return k(x, gidx)


def _sc_scatter(y, sidx):
    mesh = plsc.VectorSubcoreMesh(core_axis_name="c", subcore_axis_name="s")

    @functools.partial(
        pl.kernel,
        mesh=mesh,
        out_type=jax.ShapeDtypeStruct((T, D), jnp.float32),
        scratch_types=[
            pltpu.VMEM((BPW,), jnp.int32),
            pltpu.VMEM((BPW, D), jnp.float32),
            pltpu.SemaphoreType.DMA,
        ],
    )
    def k(y_hbm, idx_hbm, out_hbm, idx_v, rows_v, sem):
        wid = lax.axis_index("s") * 2 + lax.axis_index("c")
        base = wid * BPW
        pltpu.sync_copy(idx_hbm.at[pl.ds(base, BPW)], idx_v)
        pltpu.sync_copy(y_hbm.at[pl.ds(base, BPW)], rows_v)
        pltpu.async_copy(rows_v, out_hbm.at[idx_v], sem).wait()

    return k(y, sidx)


def _schedule(eidx):
    order = jnp.argsort(eidx).astype(jnp.int32)
    cnt = jnp.sum(eidx[None, :] == jnp.arange(E, dtype=eidx.dtype)[:, None],
                  axis=1).astype(jnp.int32)
    offs = (jnp.cumsum(cnt) - cnt).astype(jnp.int32)
    cum_end = offs + cnt

    cuts = jnp.sort(jnp.concatenate(
        [jnp.arange(NT, dtype=jnp.int32) * TILE_M, offs[1:]]))
    lo = cuts
    hi = jnp.concatenate([cuts[1:], jnp.full((1,), T, jnp.int32)])
    validv = hi > lo
    eraw = jnp.clip(jnp.searchsorted(cum_end, lo, side="right"),
                    0, E - 1).astype(jnp.int32)
    traw = jnp.clip(lo // TILE_M, 0, NT - 1)

    v = jnp.arange(STEPS, dtype=jnp.int32)
    prev = jnp.clip(lax.cummax(jnp.where(validv, v, -1)), 0, STEPS - 1)
    eof = jnp.where(validv, eraw, eraw[prev])
    tile = jnp.where(validv, traw, traw[prev])

    same_before = ((eraw[None, :] == eraw[:, None]) & validv[None, :]
                   & (v[None, :] < v[:, None]))
    dirn = (jnp.sum(same_before, axis=1) % 2).astype(jnp.int32)
    final_neff = jnp.where(dirn == 0, NB - 1, 0)
    narr = jnp.broadcast_to(jnp.arange(NB, dtype=jnp.int32)[None, :],
                            (STEPS, NB))
    nmap = jnp.where(validv[:, None],
                     jnp.where(dirn[:, None] == 1, NB - 1 - narr, narr),
                     final_neff[prev][:, None])
    first = (validv & (lo % TILE_M == 0)).astype(jnp.int32)
    valid = validv.astype(jnp.int32)
    return (tile, eof, lo, hi, valid, first, nmap), order


def kernel(x, Wr, w_v, c_proj):
    eidx = _router(x, Wr)
    sched, order = _schedule(eidx)
    xs = _sc_gather(x, order)
    y = _grouped_mlp(xs, w_v, c_proj, sched)
    return _sc_scatter(y, order)


# speedup vs baseline: 3.4483x; 1.0594x over previous
import functools

import jax
import jax.numpy as jnp
from jax import lax
from jax.experimental import pallas as pl
from jax.experimental.pallas import tpu as pltpu
from jax.experimental.pallas import tpu_sc as plsc

T = 2048
D = 768
E = 8
H = 4 * D
TILE_M = 256
NT = T // TILE_M
STEPS = NT + E - 1
NBLK = 1024
NB = H // NBLK
NW = 32
BPW = T // NW


def _router_body(x_ref, wr_ref, o_ref):
    logits = jnp.dot(x_ref[...], wr_ref[...], preferred_element_type=jnp.float32)
    m = jnp.max(logits, axis=1, keepdims=True)
    col = lax.broadcasted_iota(jnp.int32, logits.shape, 1)
    idx = jnp.min(jnp.where(logits >= m, col, E), axis=1, keepdims=True)
    o_ref[...] = jnp.broadcast_to(idx, o_ref.shape)


def _router(x, Wr):
    out = pl.pallas_call(
        _router_body,
        out_shape=jax.ShapeDtypeStruct((T, E), jnp.int32),
    )(x, Wr)
    return out[:, 0]


def _mlp_body(tile_ref, eof_ref, lo_ref, hi_ref, valid_ref, first_ref,
              nmap_ref, xs_ref, wg_ref, wu_ref, cp_ref, o_ref):
    s = pl.program_id(0)
    n = pl.program_id(1)

    @pl.when(valid_ref[s] == 1)
    def _():
        gid = tile_ref[s] * TILE_M + lax.broadcasted_iota(
            jnp.int32, (TILE_M, 1), 0)
        msk = ((gid >= lo_ref[s]) & (gid < hi_ref[s])).astype(jnp.bfloat16)
        xt = xs_ref[...].astype(jnp.bfloat16) * msk
        wg = wg_ref[0].astype(jnp.bfloat16)
        wu = wu_ref[0].astype(jnp.bfloat16)
        cp = cp_ref[0].astype(jnp.bfloat16)
        g = jnp.dot(xt, wg, preferred_element_type=jnp.float32)
        u = jnp.dot(xt, wu, preferred_element_type=jnp.float32)
        a = (g * jax.nn.sigmoid(g) * u).astype(jnp.bfloat16)
        part = jnp.dot(a, cp, preferred_element_type=jnp.float32)
        init = (n == 0) & (first_ref[s] == 1)

        @pl.when(init)
        def _():
            o_ref[...] = part

        @pl.when(jnp.logical_not(init))
        def _():
            o_ref[...] += part


def _grouped_mlp(xs, w_v, c_proj, sched):
    grid_spec = pltpu.PrefetchScalarGridSpec(
        num_scalar_prefetch=7,
        grid=(STEPS, NB),
        in_specs=[
            pl.BlockSpec((TILE_M, D),
                         lambda s, n, tile, eof, lo, hi, valid, first, nmap:
                         (tile[s], 0)),
            pl.BlockSpec((1, D, NBLK),
                         lambda s, n, tile, eof, lo, hi, valid, first, nmap:
                         (eof[s], 0, nmap[s, n])),
            pl.BlockSpec((1, D, NBLK),
                         lambda s, n, tile, eof, lo, hi, valid, first, nmap:
                         (eof[s], 0, NB + nmap[s, n])),
            pl.BlockSpec((1, NBLK, D),
                         lambda s, n, tile, eof, lo, hi, valid, first, nmap:
                         (eof[s], nmap[s, n], 0)),
        ],
        out_specs=pl.BlockSpec((TILE_M, D),
                               lambda s, n, tile, eof, lo, hi, valid, first,
                               nmap: (tile[s], 0)),
    )
    return pl.pallas_call(
        _mlp_body,
        grid_spec=grid_spec,
        out_shape=jax.ShapeDtypeStruct((T, D), jnp.float32),
        compiler_params=pltpu.CompilerParams(
            dimension_semantics=("arbitrary", "arbitrary")),
    )(*sched, xs, w_v, w_v, c_proj)


def _sc_gather(x, gidx):
    mesh = plsc.VectorSubcoreMesh(core_axis_name="c", subcore_axis_name="s")

    @functools.partial(
        pl.kernel,
        mesh=mesh,
        out_type=jax.ShapeDtypeStruct((T, D), jnp.float32),
        scratch_types=[
            pltpu.VMEM((BPW,), jnp.int32),
            pltpu.VMEM((BPW, D), jnp.float32),
            pltpu.SemaphoreType.DMA,
        ],
    )
    def k(x_hbm, idx_hbm, out_hbm, idx_v, rows_v, sem):
        wid = lax.axis_index("s") * 2 + lax.axis_index("c")
        base = wid * BPW
        pltpu.sync_copy(idx_hbm.at[pl.ds(base, BPW)], idx_v)
        pltpu.async_copy(x_hbm.at[idx_v], rows_v, sem).wait()
        pltpu.sync_copy(rows_v, out_hbm.at[pl.ds(base, BPW)])

    return k(x, gidx)


def _sc_scatter(y, sidx):
    mesh = plsc.VectorSubcoreMesh(core_axis_name="c", subcore_axis_name="s")

    @functools.partial(
        pl.kernel,
        mesh=mesh,
        out_type=jax.ShapeDtypeStruct((T, D), jnp.float32),
        scratch_types=[
            pltpu.VMEM((BPW,), jnp.int32),
            pltpu.VMEM((BPW, D), jnp.float32),
            pltpu.SemaphoreType.DMA,
        ],
    )
    def k(y_hbm, idx_hbm, out_hbm, idx_v, rows_v, sem):
        wid = lax.axis_index("s") * 2 + lax.axis_index("c")
        base = wid * BPW
        pltpu.sync_copy(idx_hbm.at[pl.ds(base, BPW)], idx_v)
        pltpu.sync_copy(y_hbm.at[pl.ds(base, BPW)], rows_v)
        pltpu.async_copy(rows_v, out_hbm.at[idx_v], sem).wait()

    return k(y, sidx)


def _schedule(eidx):
    order = jnp.argsort(eidx).astype(jnp.int32)
    cnt = jnp.sum(eidx[None, :] == jnp.arange(E, dtype=eidx.dtype)[:, None],
                  axis=1).astype(jnp.int32)
    offs = (jnp.cumsum(cnt) - cnt).astype(jnp.int32)
    cum_end = offs + cnt

    cuts = jnp.sort(jnp.concatenate(
        [jnp.arange(NT, dtype=jnp.int32) * TILE_M, offs[1:]]))
    lo = cuts
    hi = jnp.concatenate([cuts[1:], jnp.full((1,), T, jnp.int32)])
    validv = hi > lo
    eraw = jnp.clip(jnp.searchsorted(cum_end, lo, side="right"),
                    0, E - 1).astype(jnp.int32)
    traw = jnp.clip(lo // TILE_M, 0, NT - 1)

    v = jnp.arange(STEPS, dtype=jnp.int32)
    prev = jnp.clip(lax.cummax(jnp.where(validv, v, -1)), 0, STEPS - 1)
    eof = jnp.where(validv, eraw, eraw[prev])
    tile = jnp.where(validv, traw, traw[prev])

    same_before = ((eraw[None, :] == eraw[:, None]) & validv[None, :]
                   & (v[None, :] < v[:, None]))
    dirn = (jnp.sum(same_before, axis=1) % 2).astype(jnp.int32)
    final_neff = jnp.where(dirn == 0, NB - 1, 0)
    narr = jnp.broadcast_to(jnp.arange(NB, dtype=jnp.int32)[None, :],
                            (STEPS, NB))
    nmap = jnp.where(validv[:, None],
                     jnp.where(dirn[:, None] == 1, NB - 1 - narr, narr),
                     final_neff[prev][:, None])
    first = (validv & (lo % TILE_M == 0)).astype(jnp.int32)
    valid = validv.astype(jnp.int32)
    return (tile, eof, lo, hi, valid, first, nmap), order


def kernel(x, Wr, w_v, c_proj):
    eidx = _router(x, Wr)
    sched, order = _schedule(eidx)
    xs = _sc_gather(x, order)
    y = _grouped_mlp(xs, w_v, c_proj, sched)
    return _sc_scatter(y, order)


# speedup vs baseline: 3.5912x; 1.0414x over previous
import functools

import jax
import jax.numpy as jnp
from jax import lax
from jax.experimental import pallas as pl
from jax.experimental.pallas import tpu as pltpu
from jax.experimental.pallas import tpu_sc as plsc

T = 2048
D = 768
E = 8
H = 4 * D
TILE_M = 256
NT = T // TILE_M
STEPS = NT + E - 1
NBLK = 1536
NB = H // NBLK
NW = 32
BPW = T // NW


def _router_body(x_ref, wr_ref, o_ref):
    logits = jnp.dot(x_ref[...], wr_ref[...], preferred_element_type=jnp.float32)
    m = jnp.max(logits, axis=1, keepdims=True)
    col = lax.broadcasted_iota(jnp.int32, logits.shape, 1)
    idx = jnp.min(jnp.where(logits >= m, col, E), axis=1, keepdims=True)
    o_ref[...] = jnp.broadcast_to(idx, o_ref.shape)


def _router(x, Wr):
    out = pl.pallas_call(
        _router_body,
        out_shape=jax.ShapeDtypeStruct((T, E), jnp.int32),
    )(x, Wr)
    return out[:, 0]


def _mlp_body(tile_ref, eof_ref, lo_ref, hi_ref, valid_ref, first_ref,
              nmap_ref, xs_ref, wg_ref, wu_ref, cp_ref, o_ref):
    s = pl.program_id(0)
    n = pl.program_id(1)

    @pl.when(valid_ref[s] == 1)
    def _():
        gid = tile_ref[s] * TILE_M + lax.broadcasted_iota(
            jnp.int32, (TILE_M, 1), 0)
        msk = ((gid >= lo_ref[s]) & (gid < hi_ref[s])).astype(jnp.bfloat16)
        xt = xs_ref[...].astype(jnp.bfloat16) * msk
        wg = wg_ref[0].astype(jnp.bfloat16)
        wu = wu_ref[0].astype(jnp.bfloat16)
        cp = cp_ref[0].astype(jnp.bfloat16)
        g = jnp.dot(xt, wg, preferred_element_type=jnp.float32)
        u = jnp.dot(xt, wu, preferred_element_type=jnp.float32)
        a = (g * jax.nn.sigmoid(g) * u).astype(jnp.bfloat16)
        part = jnp.dot(a, cp, preferred_element_type=jnp.float32)
        init = (n == 0) & (first_ref[s] == 1)

        @pl.when(init)
        def _():
            o_ref[...] = part

        @pl.when(jnp.logical_not(init))
        def _():
            o_ref[...] += part


def _grouped_mlp(xs, w_v, c_proj, sched):
    grid_spec = pltpu.PrefetchScalarGridSpec(
        num_scalar_prefetch=7,
        grid=(STEPS, NB),
        in_specs=[
            pl.BlockSpec((TILE_M, D),
                         lambda s, n, tile, eof, lo, hi, valid, first, nmap:
                         (tile[s], 0)),
            pl.BlockSpec((1, D, NBLK),
                         lambda s, n, tile, eof, lo, hi, valid, first, nmap:
                         (eof[s], 0, nmap[s, n])),
            pl.BlockSpec((1, D, NBLK),
                         lambda s, n, tile, eof, lo, hi, valid, first, nmap:
                         (eof[s], 0, NB + nmap[s, n])),
            pl.BlockSpec((1, NBLK, D),
                         lambda s, n, tile, eof, lo, hi, valid, first, nmap:
                         (eof[s], nmap[s, n], 0)),
        ],
        out_specs=pl.BlockSpec((TILE_M, D),
                               lambda s, n, tile, eof, lo, hi, valid, first,
                               nmap: (tile[s], 0)),
    )
    return pl.pallas_call(
        _mlp_body,
        grid_spec=grid_spec,
        out_shape=jax.ShapeDtypeStruct((T, D), jnp.float32),
        compiler_params=pltpu.CompilerParams(
            dimension_semantics=("arbitrary", "arbitrary")),
    )(*sched, xs, w_v, w_v, c_proj)


def _sc_gather(x, gidx):
    mesh = plsc.VectorSubcoreMesh(core_axis_name="c", subcore_axis_name="s")

    @functools.partial(
        pl.kernel,
        mesh=mesh,
        out_type=jax.ShapeDtypeStruct((T, D), jnp.float32),
        scratch_types=[
            pltpu.VMEM((BPW,), jnp.int32),
            pltpu.VMEM((BPW, D), jnp.float32),
            pltpu.SemaphoreType.DMA,
        ],
    )
    def k(x_hbm, idx_hbm, out_hbm, idx_v, rows_v, sem):
        wid = lax.axis_index("s") * 2 + lax.axis_index("c")
        base = wid * BPW
        pltpu.sync_copy(idx_hbm.at[pl.ds(base, BPW)], idx_v)
        pltpu.async_copy(x_hbm.at[idx_v], rows_v, sem).wait()
        pltpu.sync_copy(rows_v, out_hbm.at[pl.ds(base, BPW)])

    return k(x, gidx)


def _sc_scatter(y, sidx):
    mesh = plsc.VectorSubcoreMesh(core_axis_name="c", subcore_axis_name="s")

    @functools.partial(
        pl.kernel,
        mesh=mesh,
        out_type=jax.ShapeDtypeStruct((T, D), jnp.float32),
        scratch_types=[
            pltpu.VMEM((BPW,), jnp.int32),
            pltpu.VMEM((BPW, D), jnp.float32),
            pltpu.SemaphoreType.DMA,
        ],
    )
    def k(y_hbm, idx_hbm, out_hbm, idx_v, rows_v, sem):
        wid = lax.axis_index("s") * 2 + lax.axis_index("c")
        base = wid * BPW
        pltpu.sync_copy(idx_hbm.at[pl.ds(base, BPW)], idx_v)
        pltpu.sync_copy(y_hbm.at[pl.ds(base, BPW)], rows_v)
        pltpu.async_copy(rows_v, out_hbm.at[idx_v], sem).wait()

    return k(y, sidx)


def _schedule(eidx):
    order = jnp.argsort(eidx).astype(jnp.int32)
    cnt = jnp.sum(eidx[None, :] == jnp.arange(E, dtype=eidx.dtype)[:, None],
                  axis=1).astype(jnp.int32)
    offs = (jnp.cumsum(cnt) - cnt).astype(jnp.int32)
    cum_end = offs + cnt

    cuts = jnp.sort(jnp.concatenate(
        [jnp.arange(NT, dtype=jnp.int32) * TILE_M, offs[1:]]))
    lo = cuts
    hi = jnp.concatenate([cuts[1:], jnp.full((1,), T, jnp.int32)])
    validv = hi > lo
    eraw = jnp.clip(jnp.searchsorted(cum_end, lo, side="right"),
                    0, E - 1).astype(jnp.int32)
    traw = jnp.clip(lo // TILE_M, 0, NT - 1)

    v = jnp.arange(STEPS, dtype=jnp.int32)
    prev = jnp.clip(lax.cummax(jnp.where(validv, v, -1)), 0, STEPS - 1)
    eof = jnp.where(validv, eraw, eraw[prev])
    tile = jnp.where(validv, traw, traw[prev])

    same_before = ((eraw[None, :] == eraw[:, None]) & validv[None, :]
                   & (v[None, :] < v[:, None]))
    dirn = (jnp.sum(same_before, axis=1) % 2).astype(jnp.int32)
    final_neff = jnp.where(dirn == 0, NB - 1, 0)
    narr = jnp.broadcast_to(jnp.arange(NB, dtype=jnp.int32)[None, :],
                            (STEPS, NB))
    nmap = jnp.where(validv[:, None],
                     jnp.where(dirn[:, None] == 1, NB - 1 - narr, narr),
                     final_neff[prev][:, None])
    first = (validv & (lo % TILE_M == 0)).astype(jnp.int32)
    valid = validv.astype(jnp.int32)
    return (tile, eof, lo, hi, valid, first, nmap), order


def kernel(x, Wr, w_v, c_proj):
    eidx = _router(x, Wr)
    sched, order = _schedule(eidx)
    xs = _sc_gather(x, order)
    y = _grouped_mlp(xs, w_v, c_proj, sched)
    return _sc_scatter(y, order)


# speedup vs baseline: 3.7983x; 1.0577x over previous
import functools

import jax
import jax.numpy as jnp
from jax import lax
from jax.experimental import pallas as pl
from jax.experimental.pallas import tpu as pltpu
from jax.experimental.pallas import tpu_sc as plsc

T = 2048
D = 768
E = 8
H = 4 * D
TILE_M = 128
NT = T // TILE_M
STEPS = NT + E - 1
NBLK = 3072
NB = H // NBLK
NW = 32
BPW = T // NW


def _router_body(x_ref, wr_ref, o_ref):
    logits = jnp.dot(x_ref[...], wr_ref[...], preferred_element_type=jnp.float32)
    m = jnp.max(logits, axis=1, keepdims=True)
    col = lax.broadcasted_iota(jnp.int32, logits.shape, 1)
    idx = jnp.min(jnp.where(logits >= m, col, E), axis=1, keepdims=True)
    o_ref[...] = jnp.broadcast_to(idx, o_ref.shape)


def _router(x, Wr):
    out = pl.pallas_call(
        _router_body,
        out_shape=jax.ShapeDtypeStruct((T, E), jnp.int32),
    )(x, Wr)
    return out[:, 0]


def _mlp_body(tile_ref, eof_ref, lo_ref, hi_ref, valid_ref, first_ref,
              nmap_ref, xs_ref, wg_ref, wu_ref, cp_ref, o_ref):
    s = pl.program_id(0)
    n = pl.program_id(1)

    @pl.when(valid_ref[s] == 1)
    def _():
        gid = tile_ref[s] * TILE_M + lax.broadcasted_iota(
            jnp.int32, (TILE_M, 1), 0)
        msk = ((gid >= lo_ref[s]) & (gid < hi_ref[s])).astype(jnp.bfloat16)
        xt = xs_ref[...].astype(jnp.bfloat16) * msk
        wg = wg_ref[0].astype(jnp.bfloat16)
        wu = wu_ref[0].astype(jnp.bfloat16)
        cp = cp_ref[0].astype(jnp.bfloat16)
        g = jnp.dot(xt, wg, preferred_element_type=jnp.float32)
        u = jnp.dot(xt, wu, preferred_element_type=jnp.float32)
        a = (g * jax.nn.sigmoid(g) * u).astype(jnp.bfloat16)
        part = jnp.dot(a, cp, preferred_element_type=jnp.float32)
        init = (n == 0) & (first_ref[s] == 1)

        @pl.when(init)
        def _():
            o_ref[...] = part

        @pl.when(jnp.logical_not(init))
        def _():
            o_ref[...] += part


def _grouped_mlp(xs, w_v, c_proj, sched):
    grid_spec = pltpu.PrefetchScalarGridSpec(
        num_scalar_prefetch=7,
        grid=(STEPS, NB),
        in_specs=[
            pl.BlockSpec((TILE_M, D),
                         lambda s, n, tile, eof, lo, hi, valid, first, nmap:
                         (tile[s], 0)),
            pl.BlockSpec((1, D, NBLK),
                         lambda s, n, tile, eof, lo, hi, valid, first, nmap:
                         (eof[s], 0, nmap[s, n])),
            pl.BlockSpec((1, D, NBLK),
                         lambda s, n, tile, eof, lo, hi, valid, first, nmap:
                         (eof[s], 0, NB + nmap[s, n])),
            pl.BlockSpec((1, NBLK, D),
                         lambda s, n, tile, eof, lo, hi, valid, first, nmap:
                         (eof[s], nmap[s, n], 0)),
        ],
        out_specs=pl.BlockSpec((TILE_M, D),
                               lambda s, n, tile, eof, lo, hi, valid, first,
                               nmap: (tile[s], 0)),
    )
    return pl.pallas_call(
        _mlp_body,
        grid_spec=grid_spec,
        out_shape=jax.ShapeDtypeStruct((T, D), jnp.float32),
        compiler_params=pltpu.CompilerParams(
            dimension_semantics=("arbitrary", "arbitrary")),
    )(*sched, xs, w_v, w_v, c_proj)


def _sc_gather(x, gidx):
    mesh = plsc.VectorSubcoreMesh(core_axis_name="c", subcore_axis_name="s")

    @functools.partial(
        pl.kernel,
        mesh=mesh,
        out_type=jax.ShapeDtypeStruct((T, D), jnp.float32),
        scratch_types=[
            pltpu.VMEM((BPW,), jnp.int32),
            pltpu.VMEM((BPW, D), jnp.float32),
            pltpu.SemaphoreType.DMA,
        ],
    )
    def k(x_hbm, idx_hbm, out_hbm, idx_v, rows_v, sem):
        wid = lax.axis_index("s") * 2 + lax.axis_index("c")
        base = wid * BPW
        pltpu.sync_copy(idx_hbm.at[pl.ds(base, BPW)], idx_v)
        pltpu.async_copy(x_hbm.at[idx_v], rows_v, sem).wait()
        pltpu.sync_copy(rows_v, out_hbm.at[pl.ds(base, BPW)])

    return k(x, gidx)


def _sc_scatter(y, sidx):
    mesh = plsc.VectorSubcoreMesh(core_axis_name="c", subcore_axis_name="s")

    @functools.partial(
        pl.kernel,
        mesh=mesh,
        out_type=jax.ShapeDtypeStruct((T, D), jnp.float32),
        scratch_types=[
            pltpu.VMEM((BPW,), jnp.int32),
            pltpu.VMEM((BPW, D), jnp.float32),
            pltpu.SemaphoreType.DMA,
        ],
    )
    def k(y_hbm, idx_hbm, out_hbm, idx_v, rows_v, sem):
        wid = lax.axis_index("s") * 2 + lax.axis_index("c")
        base = wid * BPW
        pltpu.sync_copy(idx_hbm.at[pl.ds(base, BPW)], idx_v)
        pltpu.sync_copy(y_hbm.at[pl.ds(base, BPW)], rows_v)
        pltpu.async_copy(rows_v, out_hbm.at[idx_v], sem).wait()

    return k(y, sidx)


def _schedule(eidx):
    order = jnp.argsort(eidx).astype(jnp.int32)
    cnt = jnp.sum(eidx[None, :] == jnp.arange(E, dtype=eidx.dtype)[:, None],
                  axis=1).astype(jnp.int32)
    offs = (jnp.cumsum(cnt) - cnt).astype(jnp.int32)
    cum_end = offs + cnt

    cuts = jnp.sort(jnp.concatenate(
        [jnp.arange(NT, dtype=jnp.int32) * TILE_M, offs[1:]]))
    lo = cuts
    hi = jnp.concatenate([cuts[1:], jnp.full((1,), T, jnp.int32)])
    validv = hi > lo
    eraw = jnp.clip(jnp.searchsorted(cum_end, lo, side="right"),
                    0, E - 1).astype(jnp.int32)
    traw = jnp.clip(lo // TILE_M, 0, NT - 1)

    v = jnp.arange(STEPS, dtype=jnp.int32)
    prev = jnp.clip(lax.cummax(jnp.where(validv, v, -1)), 0, STEPS - 1)
    eof = jnp.where(validv, eraw, eraw[prev])
    tile = jnp.where(validv, traw, traw[prev])

    same_before = ((eraw[None, :] == eraw[:, None]) & validv[None, :]
                   & (v[None, :] < v[:, None]))
    dirn = (jnp.sum(same_before, axis=1) % 2).astype(jnp.int32)
    final_neff = jnp.where(dirn == 0, NB - 1, 0)
    narr = jnp.broadcast_to(jnp.arange(NB, dtype=jnp.int32)[None, :],
                            (STEPS, NB))
    nmap = jnp.where(validv[:, None],
                     jnp.where(dirn[:, None] == 1, NB - 1 - narr, narr),
                     final_neff[prev][:, None])
    first = (validv & (lo % TILE_M == 0)).astype(jnp.int32)
    valid = validv.astype(jnp.int32)
    return (tile, eof, lo, hi, valid, first, nmap), order


def kernel(x, Wr, w_v, c_proj):
    eidx = _router(x, Wr)
    sched, order = _schedule(eidx)
    xs = _sc_gather(x, order)
    y = _grouped_mlp(xs, w_v, c_proj, sched)
    return _sc_scatter(y, order)
